# baseline (speedup 1.0000x reference)
import jax
import jax.numpy as jnp
from jax.experimental import pallas as pl
from jax.experimental.pallas import tpu as pltpu

EPS = 1e-7
TILE = 256


def _zero_col0(m):
    col = jax.lax.broadcasted_iota(jnp.int32, m.shape, 1)
    return jnp.where(col == 0, 0.0, m)


def _roundtrip(u):
    n2 = jnp.sum(u * u, axis=-1, keepdims=True)
    n = jnp.sqrt(jnp.maximum(n2, EPS))
    e = jnp.exp(n)
    ei = jnp.exp(-n)
    sinh_n = (e - ei) * 0.5
    cosh_n = (e + ei) * 0.5
    sp = (sinh_n / n) * u
    t = jnp.maximum(cosh_n, 1.0 + EPS)
    r = jnp.log(t + jnp.sqrt(t + 1.0) * jnp.sqrt(t - 1.0))
    ns2 = jnp.sum(sp * sp, axis=-1, keepdims=True)
    ns = jnp.sqrt(jnp.maximum(ns2, EPS))
    return (r / ns) * sp


def _embed_kernel(x_ref, we_ref, w0_ref, b0_ref, g1_ref):
    h = jnp.dot(x_ref[0], we_ref[...], preferred_element_type=jnp.float32)
    h = _zero_col0(h)
    g1_ref[0] = (
        jnp.dot(h, w0_ref[...], preferred_element_type=jnp.float32) + b0_ref[...]
    )


def _layer1_kernel(adj_ref, g1_ref, w1_ref, b1_ref, g2_ref):
    a = jnp.dot(adj_ref[0], g1_ref[0], preferred_element_type=jnp.float32)
    u = _zero_col0(jnp.maximum(a, 0.0))
    v = _roundtrip(u)
    g2_ref[0] = (
        jnp.dot(v, w1_ref[...], preferred_element_type=jnp.float32) + b1_ref[...]
    )


def _layer2_kernel(adj_ref, g2_ref, wml_ref, bml_ref, mask_ref, out_ref):
    a = jnp.dot(adj_ref[0], g2_ref[0], preferred_element_type=jnp.float32)
    u = _zero_col0(jnp.maximum(a, 0.0))
    v = _roundtrip(u)
    ml = jnp.dot(v, wml_ref[...], preferred_element_type=jnp.float32) + bml_ref[...]
    out_ref[0] = ml * mask_ref[0]


def kernel(x, adj, node_mask, W_emb, W0, b0, W1, b1, W_ml, b_ml):
    B, N, F_IN = x.shape
    H = W0.shape[0]
    OUT = W_ml.shape[1]
    NT = N // TILE
    b0r = b0.reshape(1, H)
    b1r = b1.reshape(1, H)
    bmlr = b_ml.reshape(1, OUT)

    g1 = pl.pallas_call(
        _embed_kernel,
        grid=(B,),
        in_specs=[
            pl.BlockSpec((1, N, F_IN), lambda b: (b, 0, 0)),
            pl.BlockSpec((F_IN, H), lambda b: (0, 0)),
            pl.BlockSpec((H, H), lambda b: (0, 0)),
            pl.BlockSpec((1, H), lambda b: (0, 0)),
        ],
        out_specs=pl.BlockSpec((1, N, H), lambda b: (b, 0, 0)),
        out_shape=jax.ShapeDtypeStruct((B, N, H), jnp.float32),
    )(x, W_emb, W0, b0r)

    g2 = pl.pallas_call(
        _layer1_kernel,
        grid=(B, NT),
        in_specs=[
            pl.BlockSpec((1, TILE, N), lambda b, i: (b, i, 0)),
            pl.BlockSpec((1, N, H), lambda b, i: (b, 0, 0)),
            pl.BlockSpec((H, H), lambda b, i: (0, 0)),
            pl.BlockSpec((1, H), lambda b, i: (0, 0)),
        ],
        out_specs=pl.BlockSpec((1, TILE, H), lambda b, i: (b, i, 0)),
        out_shape=jax.ShapeDtypeStruct((B, N, H), jnp.float32),
        compiler_params=pltpu.CompilerParams(
            dimension_semantics=("parallel", "parallel"),
        ),
    )(adj, g1, W1, b1r)

    out = pl.pallas_call(
        _layer2_kernel,
        grid=(B, NT),
        in_specs=[
            pl.BlockSpec((1, TILE, N), lambda b, i: (b, i, 0)),
            pl.BlockSpec((1, N, H), lambda b, i: (b, 0, 0)),
            pl.BlockSpec((H, OUT), lambda b, i: (0, 0)),
            pl.BlockSpec((1, OUT), lambda b, i: (0, 0)),
            pl.BlockSpec((1, TILE, 1), lambda b, i: (b, i, 0)),
        ],
        out_specs=pl.BlockSpec((1, TILE, OUT), lambda b, i: (b, i, 0)),
        out_shape=jax.ShapeDtypeStruct((B, N, OUT), jnp.float32),
        compiler_params=pltpu.CompilerParams(
            dimension_semantics=("parallel", "parallel"),
        ),
    )(adj, g2, W_ml, bmlr, node_mask)

    return out


# speedup vs baseline: 1.0185x; 1.0185x over previous
import jax
import jax.numpy as jnp
from jax.experimental import pallas as pl
from jax.experimental.pallas import tpu as pltpu

EPS = 1e-7
TILE = 256


def _zero_col0(m):
    col = jax.lax.broadcasted_iota(jnp.int32, m.shape, 1)
    return jnp.where(col == 0, 0.0, m)


def _roundtrip(u):
    n2 = jnp.sum(u * u, axis=-1, keepdims=True)
    n = jnp.sqrt(jnp.maximum(n2, EPS))
    e = jnp.exp(n)
    ei = jnp.exp(-n)
    sinh_n = (e - ei) * 0.5
    cosh_n = (e + ei) * 0.5
    sp = (sinh_n / n) * u
    t = jnp.maximum(cosh_n, 1.0 + EPS)
    r = jnp.log(t + jnp.sqrt(t + 1.0) * jnp.sqrt(t - 1.0))
    ns2 = jnp.sum(sp * sp, axis=-1, keepdims=True)
    ns = jnp.sqrt(jnp.maximum(ns2, EPS))
    return (r / ns) * sp


def _bdot(a, b):
    return jnp.dot(
        a.astype(jnp.bfloat16),
        b.astype(jnp.bfloat16),
        preferred_element_type=jnp.float32,
    )


def _embed_kernel(x_ref, we_ref, w0_ref, b0_ref, g1_ref):
    h = _bdot(x_ref[0], we_ref[...])
    h = _zero_col0(h)
    g1_ref[0] = _bdot(h, w0_ref[...]) + b0_ref[...]


def _layer1_kernel(adj_ref, g1_ref, w1_ref, b1_ref, g2_ref):
    a = _bdot(adj_ref[0], g1_ref[0])
    u = _zero_col0(jnp.maximum(a, 0.0))
    v = _roundtrip(u)
    g2_ref[0] = _bdot(v, w1_ref[...]) + b1_ref[...]


def _layer2_kernel(adj_ref, g2_ref, wml_ref, bml_ref, mask_ref, out_ref):
    a = _bdot(adj_ref[0], g2_ref[0])
    u = _zero_col0(jnp.maximum(a, 0.0))
    v = _roundtrip(u)
    out_ref[0] = (_bdot(v, wml_ref[...]) + bml_ref[...]) * mask_ref[0]


def kernel(x, adj, node_mask, W_emb, W0, b0, W1, b1, W_ml, b_ml):
    B, N, F_IN = x.shape
    H = W0.shape[0]
    OUT = W_ml.shape[1]
    NT = N // TILE
    b0r = b0.reshape(1, H)
    b1r = b1.reshape(1, H)
    bmlr = b_ml.reshape(1, OUT)

    g1 = pl.pallas_call(
        _embed_kernel,
        grid=(B,),
        in_specs=[
            pl.BlockSpec((1, N, F_IN), lambda b: (b, 0, 0)),
            pl.BlockSpec((F_IN, H), lambda b: (0, 0)),
            pl.BlockSpec((H, H), lambda b: (0, 0)),
            pl.BlockSpec((1, H), lambda b: (0, 0)),
        ],
        out_specs=pl.BlockSpec((1, N, H), lambda b: (b, 0, 0)),
        out_shape=jax.ShapeDtypeStruct((B, N, H), jnp.float32),
    )(x, W_emb, W0, b0r)

    g2 = pl.pallas_call(
        _layer1_kernel,
        grid=(B, NT),
        in_specs=[
            pl.BlockSpec((1, TILE, N), lambda b, i: (b, i, 0)),
            pl.BlockSpec((1, N, H), lambda b, i: (b, 0, 0)),
            pl.BlockSpec((H, H), lambda b, i: (0, 0)),
            pl.BlockSpec((1, H), lambda b, i: (0, 0)),
        ],
        out_specs=pl.BlockSpec((1, TILE, H), lambda b, i: (b, i, 0)),
        out_shape=jax.ShapeDtypeStruct((B, N, H), jnp.float32),
        compiler_params=pltpu.CompilerParams(
            dimension_semantics=("parallel", "parallel"),
        ),
    )(adj, g1, W1, b1r)

    out = pl.pallas_call(
        _layer2_kernel,
        grid=(B, NT),
        in_specs=[
            pl.BlockSpec((1, TILE, N), lambda b, i: (b, i, 0)),
            pl.BlockSpec((1, N, H), lambda b, i: (b, 0, 0)),
            pl.BlockSpec((H, OUT), lambda b, i: (0, 0)),
            pl.BlockSpec((1, OUT), lambda b, i: (0, 0)),
            pl.BlockSpec((1, TILE, 1), lambda b, i: (b, i, 0)),
        ],
        out_specs=pl.BlockSpec((1, TILE, OUT), lambda b, i: (b, i, 0)),
        out_shape=jax.ShapeDtypeStruct((B, N, OUT), jnp.float32),
        compiler_params=pltpu.CompilerParams(
            dimension_semantics=("parallel", "parallel"),
        ),
    )(adj, g2, W_ml, bmlr, node_mask)

    return out


# speedup vs baseline: 1.5481x; 1.5200x over previous
import jax
import jax.numpy as jnp
from jax.experimental import pallas as pl
from jax.experimental.pallas import tpu as pltpu

EPS = 1e-7
TILE = 1024


def _zero_col0(m):
    col = jax.lax.broadcasted_iota(jnp.int32, m.shape, 1)
    return jnp.where(col == 0, 0.0, m)


def _roundtrip(u):
    n2 = jnp.sum(u * u, axis=-1, keepdims=True)
    n = jnp.sqrt(jnp.maximum(n2, EPS))
    e = jnp.exp(n)
    ei = jnp.exp(-n)
    sinh_n = (e - ei) * 0.5
    cosh_n = (e + ei) * 0.5
    sp = (sinh_n / n) * u
    t = jnp.maximum(cosh_n, 1.0 + EPS)
    r = jnp.log(t + jnp.sqrt(t + 1.0) * jnp.sqrt(t - 1.0))
    ns2 = jnp.sum(sp * sp, axis=-1, keepdims=True)
    ns = jnp.sqrt(jnp.maximum(ns2, EPS))
    return (r / ns) * sp


def _bdot(a, b):
    return jnp.dot(
        a.astype(jnp.bfloat16),
        b.astype(jnp.bfloat16),
        preferred_element_type=jnp.float32,
    )


def _embed_kernel(x_ref, we_ref, w0_ref, b0_ref, g1_ref):
    h = _bdot(x_ref[0], we_ref[...])
    h = _zero_col0(h)
    g1_ref[0] = _bdot(h, w0_ref[...]) + b0_ref[...]


def _layer1_kernel(adj_ref, g1_ref, w1_ref, b1_ref, g2_ref):
    a = _bdot(adj_ref[0], g1_ref[0])
    u = _zero_col0(jnp.maximum(a, 0.0))
    v = _roundtrip(u)
    g2_ref[0] = _bdot(v, w1_ref[...]) + b1_ref[...]


def _layer2_kernel(adj_ref, g2_ref, wml_ref, bml_ref, mask_ref, out_ref):
    a = _bdot(adj_ref[0], g2_ref[0])
    u = _zero_col0(jnp.maximum(a, 0.0))
    v = _roundtrip(u)
    out_ref[0] = (_bdot(v, wml_ref[...]) + bml_ref[...]) * mask_ref[0]


def kernel(x, adj, node_mask, W_emb, W0, b0, W1, b1, W_ml, b_ml):
    B, N, F_IN = x.shape
    H = W0.shape[0]
    OUT = W_ml.shape[1]
    NT = N // TILE
    b0r = b0.reshape(1, H)
    b1r = b1.reshape(1, H)
    bmlr = b_ml.reshape(1, OUT)

    g1 = pl.pallas_call(
        _embed_kernel,
        grid=(B,),
        in_specs=[
            pl.BlockSpec((1, N, F_IN), lambda b: (b, 0, 0)),
            pl.BlockSpec((F_IN, H), lambda b: (0, 0)),
            pl.BlockSpec((H, H), lambda b: (0, 0)),
            pl.BlockSpec((1, H), lambda b: (0, 0)),
        ],
        out_specs=pl.BlockSpec((1, N, H), lambda b: (b, 0, 0)),
        out_shape=jax.ShapeDtypeStruct((B, N, H), jnp.float32),
    )(x, W_emb, W0, b0r)

    g2 = pl.pallas_call(
        _layer1_kernel,
        grid=(B, NT),
        in_specs=[
            pl.BlockSpec((1, TILE, N), lambda b, i: (b, i, 0)),
            pl.BlockSpec((1, N, H), lambda b, i: (b, 0, 0)),
            pl.BlockSpec((H, H), lambda b, i: (0, 0)),
            pl.BlockSpec((1, H), lambda b, i: (0, 0)),
        ],
        out_specs=pl.BlockSpec((1, TILE, H), lambda b, i: (b, i, 0)),
        out_shape=jax.ShapeDtypeStruct((B, N, H), jnp.float32),
        compiler_params=pltpu.CompilerParams(
            dimension_semantics=("parallel", "parallel"),
        ),
    )(adj, g1, W1, b1r)

    out = pl.pallas_call(
        _layer2_kernel,
        grid=(B, NT),
        in_specs=[
            pl.BlockSpec((1, TILE, N), lambda b, i: (b, i, 0)),
            pl.BlockSpec((1, N, H), lambda b, i: (b, 0, 0)),
            pl.BlockSpec((H, OUT), lambda b, i: (0, 0)),
            pl.BlockSpec((1, OUT), lambda b, i: (0, 0)),
            pl.BlockSpec((1, TILE, 1), lambda b, i: (b, i, 0)),
        ],
        out_specs=pl.BlockSpec((1, TILE, OUT), lambda b, i: (b, i, 0)),
        out_shape=jax.ShapeDtypeStruct((B, N, OUT), jnp.float32),
        compiler_params=pltpu.CompilerParams(
            dimension_semantics=("parallel", "parallel"),
        ),
    )(adj, g2, W_ml, bmlr, node_mask)

    return out


# speedup vs baseline: 1.6470x; 1.0639x over previous
import jax
import jax.numpy as jnp
from jax.experimental import pallas as pl
from jax.experimental.pallas import tpu as pltpu

EPS = 1e-7
TILE = 2048


def _zero_col0(m):
    col = jax.lax.broadcasted_iota(jnp.int32, m.shape, 1)
    return jnp.where(col == 0, 0.0, m)


def _roundtrip(u):
    n2 = jnp.sum(u * u, axis=-1, keepdims=True)
    n = jnp.sqrt(jnp.maximum(n2, EPS))
    e = jnp.exp(n)
    ei = jnp.exp(-n)
    sinh_n = (e - ei) * 0.5
    cosh_n = (e + ei) * 0.5
    sp = (sinh_n / n) * u
    t = jnp.maximum(cosh_n, 1.0 + EPS)
    r = jnp.log(t + jnp.sqrt(t + 1.0) * jnp.sqrt(t - 1.0))
    ns2 = jnp.sum(sp * sp, axis=-1, keepdims=True)
    ns = jnp.sqrt(jnp.maximum(ns2, EPS))
    return (r / ns) * sp


def _bdot(a, b):
    return jnp.dot(
        a.astype(jnp.bfloat16),
        b.astype(jnp.bfloat16),
        preferred_element_type=jnp.float32,
    )


def _embed_kernel(x_ref, we_ref, w0_ref, b0_ref, g1_ref):
    h = _bdot(x_ref[0], we_ref[...])
    h = _zero_col0(h)
    g1_ref[0] = _bdot(h, w0_ref[...]) + b0_ref[...]


def _layer1_kernel(adj_ref, g1_ref, w1_ref, b1_ref, g2_ref):
    a = _bdot(adj_ref[0], g1_ref[0])
    u = _zero_col0(jnp.maximum(a, 0.0))
    v = _roundtrip(u)
    g2_ref[0] = _bdot(v, w1_ref[...]) + b1_ref[...]


def _layer2_kernel(adj_ref, g2_ref, wml_ref, bml_ref, mask_ref, out_ref):
    a = _bdot(adj_ref[0], g2_ref[0])
    u = _zero_col0(jnp.maximum(a, 0.0))
    v = _roundtrip(u)
    out_ref[0] = (_bdot(v, wml_ref[...]) + bml_ref[...]) * mask_ref[0]


def kernel(x, adj, node_mask, W_emb, W0, b0, W1, b1, W_ml, b_ml):
    B, N, F_IN = x.shape
    H = W0.shape[0]
    OUT = W_ml.shape[1]
    NT = N // TILE
    b0r = b0.reshape(1, H)
    b1r = b1.reshape(1, H)
    bmlr = b_ml.reshape(1, OUT)

    g1 = pl.pallas_call(
        _embed_kernel,
        grid=(B,),
        in_specs=[
            pl.BlockSpec((1, N, F_IN), lambda b: (b, 0, 0)),
            pl.BlockSpec((F_IN, H), lambda b: (0, 0)),
            pl.BlockSpec((H, H), lambda b: (0, 0)),
            pl.BlockSpec((1, H), lambda b: (0, 0)),
        ],
        out_specs=pl.BlockSpec((1, N, H), lambda b: (b, 0, 0)),
        out_shape=jax.ShapeDtypeStruct((B, N, H), jnp.float32),
    )(x, W_emb, W0, b0r)

    g2 = pl.pallas_call(
        _layer1_kernel,
        grid=(B, NT),
        in_specs=[
            pl.BlockSpec((1, TILE, N), lambda b, i: (b, i, 0)),
            pl.BlockSpec((1, N, H), lambda b, i: (b, 0, 0)),
            pl.BlockSpec((H, H), lambda b, i: (0, 0)),
            pl.BlockSpec((1, H), lambda b, i: (0, 0)),
        ],
        out_specs=pl.BlockSpec((1, TILE, H), lambda b, i: (b, i, 0)),
        out_shape=jax.ShapeDtypeStruct((B, N, H), jnp.float32),
        compiler_params=pltpu.CompilerParams(
            dimension_semantics=("parallel", "parallel"),
        ),
    )(adj, g1, W1, b1r)

    out = pl.pallas_call(
        _layer2_kernel,
        grid=(B, NT),
        in_specs=[
            pl.BlockSpec((1, TILE, N), lambda b, i: (b, i, 0)),
            pl.BlockSpec((1, N, H), lambda b, i: (b, 0, 0)),
            pl.BlockSpec((H, OUT), lambda b, i: (0, 0)),
            pl.BlockSpec((1, OUT), lambda b, i: (0, 0)),
            pl.BlockSpec((1, TILE, 1), lambda b, i: (b, i, 0)),
        ],
        out_specs=pl.BlockSpec((1, TILE, OUT), lambda b, i: (b, i, 0)),
        out_shape=jax.ShapeDtypeStruct((B, N, OUT), jnp.float32),
        compiler_params=pltpu.CompilerParams(
            dimension_semantics=("parallel", "parallel"),
        ),
    )(adj, g2, W_ml, bmlr, node_mask)

    return out


# speedup vs baseline: 1.8072x; 1.0973x over previous
import jax
import jax.numpy as jnp
from jax.experimental import pallas as pl
from jax.experimental.pallas import tpu as pltpu

EPS = 1e-7
TILE = 2048


def _zero_col0(m):
    col = jax.lax.broadcasted_iota(jnp.int32, m.shape, 1)
    return jnp.where(col == 0, 0.0, m)


def _roundtrip(u):
    n2 = jnp.sum(u * u, axis=-1, keepdims=True)
    n = jnp.sqrt(jnp.maximum(n2, EPS))
    e = jnp.exp(n)
    ei = jnp.exp(-n)
    sinh_n = (e - ei) * 0.5
    cosh_n = (e + ei) * 0.5
    sp = (sinh_n / n) * u
    t = jnp.maximum(cosh_n, 1.0 + EPS)
    r = jnp.log(t + jnp.sqrt(t + 1.0) * jnp.sqrt(t - 1.0))
    ns2 = jnp.sum(sp * sp, axis=-1, keepdims=True)
    ns = jnp.sqrt(jnp.maximum(ns2, EPS))
    return (r / ns) * sp


def _bdot(a, b):
    return jnp.dot(
        a.astype(jnp.bfloat16),
        b.astype(jnp.bfloat16),
        preferred_element_type=jnp.float32,
    )


def _layer1_kernel(adj_ref, x_ref, we_ref, w0_ref, b0_ref, w1_ref, b1_ref, g2_ref):
    h = _bdot(x_ref[0], we_ref[...])
    h = _zero_col0(h)
    g1 = _bdot(h, w0_ref[...]) + b0_ref[...]
    a = _bdot(adj_ref[0], g1)
    u = _zero_col0(jnp.maximum(a, 0.0))
    v = _roundtrip(u)
    g2_ref[0] = (_bdot(v, w1_ref[...]) + b1_ref[...]).astype(jnp.bfloat16)


def _layer2_kernel(adj_ref, g2_ref, wml_ref, bml_ref, mask_ref, out_ref):
    a = jnp.dot(
        adj_ref[0].astype(jnp.bfloat16),
        g2_ref[0],
        preferred_element_type=jnp.float32,
    )
    u = _zero_col0(jnp.maximum(a, 0.0))
    v = _roundtrip(u)
    out_ref[0] = (_bdot(v, wml_ref[...]) + bml_ref[...]) * mask_ref[0]


def kernel(x, adj, node_mask, W_emb, W0, b0, W1, b1, W_ml, b_ml):
    B, N, F_IN = x.shape
    H = W0.shape[0]
    OUT = W_ml.shape[1]
    NT = N // TILE
    b0r = b0.reshape(1, H)
    b1r = b1.reshape(1, H)
    bmlr = b_ml.reshape(1, OUT)

    g2 = pl.pallas_call(
        _layer1_kernel,
        grid=(B, NT),
        in_specs=[
            pl.BlockSpec((1, TILE, N), lambda b, i: (b, i, 0)),
            pl.BlockSpec((1, N, F_IN), lambda b, i: (b, 0, 0)),
            pl.BlockSpec((F_IN, H), lambda b, i: (0, 0)),
            pl.BlockSpec((H, H), lambda b, i: (0, 0)),
            pl.BlockSpec((1, H), lambda b, i: (0, 0)),
            pl.BlockSpec((H, H), lambda b, i: (0, 0)),
            pl.BlockSpec((1, H), lambda b, i: (0, 0)),
        ],
        out_specs=pl.BlockSpec((1, TILE, H), lambda b, i: (b, i, 0)),
        out_shape=jax.ShapeDtypeStruct((B, N, H), jnp.bfloat16),
        compiler_params=pltpu.CompilerParams(
            dimension_semantics=("parallel", "parallel"),
        ),
    )(adj, x, W_emb, W0, b0r, W1, b1r)

    out = pl.pallas_call(
        _layer2_kernel,
        grid=(B, NT),
        in_specs=[
            pl.BlockSpec((1, TILE, N), lambda b, i: (b, i, 0)),
            pl.BlockSpec((1, N, H), lambda b, i: (b, 0, 0)),
            pl.BlockSpec((H, OUT), lambda b, i: (0, 0)),
            pl.BlockSpec((1, OUT), lambda b, i: (0, 0)),
            pl.BlockSpec((1, TILE, 1), lambda b, i: (b, i, 0)),
        ],
        out_specs=pl.BlockSpec((1, TILE, OUT), lambda b, i: (b, i, 0)),
        out_shape=jax.ShapeDtypeStruct((B, N, OUT), jnp.float32),
        compiler_params=pltpu.CompilerParams(
            dimension_semantics=("parallel", "parallel"),
        ),
    )(adj, g2, W_ml, bmlr, node_mask)

    return out
